# Initial kernel scaffold; baseline (speedup 1.0000x reference)
import jax
import jax.numpy as jnp
from jax.experimental import pallas as pl


def kernel(x, edge_index, edge_attr, batch, We1, be1, W11, b11, W12, b12, We2, be2, W21, b21, W22, b22, Wm1, bm1, Wm2, bm2):
    raise NotImplementedError("write your pallas kernel here")


# speedup vs baseline: 2.8035x; 2.8035x over previous
import functools

import jax
import jax.numpy as jnp
from jax import lax
from jax.experimental import pallas as pl
from jax.experimental.pallas import tpu as pltpu
from jax.experimental.pallas import tpu_sc as plsc

N = 10000
NE = 320000
D = 128
DE = 16
G = 128
H2 = 64
NOUT = 10

NC, NS = 2, 16
NW = NC * NS
EPW = NE // NW
C = 80
NCHUNK = EPW // C
ROWS_A = 632
ROWS_B = N - 15 * ROWS_A


def _edge_mm_body(ea, we1, be1, we2, be2, o1, o2):
    a = ea[...]
    o1[...] = jnp.dot(a, we1[...], preferred_element_type=jnp.float32) + be1[...]
    o2[...] = jnp.dot(a, we2[...], preferred_element_type=jnp.float32) + be2[...]


def _edge_transform(ea, we1, be1, we2, be2):
    br = 2000
    return pl.pallas_call(
        _edge_mm_body,
        grid=(NE // br,),
        in_specs=[
            pl.BlockSpec((br, DE), lambda i: (i, 0)),
            pl.BlockSpec((DE, D), lambda i: (0, 0)),
            pl.BlockSpec((1, D), lambda i: (0, 0)),
            pl.BlockSpec((DE, D), lambda i: (0, 0)),
            pl.BlockSpec((1, D), lambda i: (0, 0)),
        ],
        out_specs=[pl.BlockSpec((br, D), lambda i: (i, 0))] * 2,
        out_shape=[jax.ShapeDtypeStruct((NE, D), jnp.float32)] * 2,
    )(ea, we1, be1.reshape(1, D), we2, be2.reshape(1, D))


def _sc_agg_body(x_hbm, e_hbm, src_hbm, dst_hbm, out_hbm,
                 idx_s, idx_d, xb, eb, acc, sem):
    cid = lax.axis_index("c")
    sid = lax.axis_index("s")

    zero = jnp.zeros((16,), jnp.float32)

    def zrow(i, _):
        for k in range(8):
            xb[i, pl.ds(k * 16, 16)] = zero
        return 0
    lax.fori_loop(0, C, zrow, 0)

    row0 = sid * ROWS_A

    def _zero_slice(total):
        nfull, rem = total // C, total % C

        def zcp(j, _):
            pltpu.sync_copy(xb, acc.at[pl.ds(row0 + j * C, C)])
            return 0
        lax.fori_loop(0, nfull, zcp, 0)
        if rem:
            pltpu.sync_copy(xb.at[pl.ds(0, rem)],
                            acc.at[pl.ds(row0 + nfull * C, rem)])

    @pl.when(sid < NS - 1)
    def _za():
        _zero_slice(ROWS_A)

    @pl.when(sid == NS - 1)
    def _zb():
        _zero_slice(ROWS_B)
    plsc.subcore_barrier()

    ebase = (cid * NS + sid) * EPW

    def chunk(ci, _):
        base = ebase + ci * C
        pltpu.sync_copy(src_hbm.at[pl.ds(base, C)], idx_s)
        pltpu.sync_copy(dst_hbm.at[pl.ds(base, C)], idx_d)
        cp = pltpu.async_copy(x_hbm.at[idx_s], xb, sem)
        pltpu.sync_copy(e_hbm.at[pl.ds(base, C)], eb)
        cp.wait()

        def crow(i, _):
            for k in range(8):
                s = pl.ds(k * 16, 16)
                xb[i, s] = jnp.maximum(xb[i, s] + eb[i, s], 0.0)
            return 0
        lax.fori_loop(0, C, crow, 0)

        pltpu.sync_copy(xb, acc.at[idx_d], add=True)
        return 0
    lax.fori_loop(0, NCHUNK, chunk, 0)

    plsc.subcore_barrier()

    @pl.when(sid < NS - 1)
    def _oa():
        pltpu.sync_copy(acc.at[pl.ds(row0, ROWS_A)],
                        out_hbm.at[pl.ds(cid * N + row0, ROWS_A)])

    @pl.when(sid == NS - 1)
    def _ob():
        pltpu.sync_copy(acc.at[pl.ds(row0, ROWS_B)],
                        out_hbm.at[pl.ds(cid * N + row0, ROWS_B)])


@functools.cache
def _make_agg():
    return pl.kernel(
        _sc_agg_body,
        out_type=jax.ShapeDtypeStruct((2 * N, D), jnp.float32),
        mesh=plsc.VectorSubcoreMesh(core_axis_name="c", subcore_axis_name="s",
                                    num_cores=NC, num_subcores=NS),
        scratch_types=[
            pltpu.VMEM((C,), jnp.int32),
            pltpu.VMEM((C,), jnp.int32),
            pltpu.VMEM((C, D), jnp.float32),
            pltpu.VMEM((C, D), jnp.float32),
            pltpu.VMEM_SHARED((N, D), jnp.float32),
            pltpu.SemaphoreType.DMA,
        ],
    )


def _agg(x, e, src, dst):
    return _make_agg()(x, e, src, dst)


def _mlp_body(x, a0, a1, w1, b1, w2, b2, out):
    h = x[...] + a0[...] + a1[...]
    t = jnp.maximum(jnp.dot(h, w1[...], preferred_element_type=jnp.float32)
                    + b1[...], 0.0)
    o = jnp.dot(t, w2[...], preferred_element_type=jnp.float32) + b2[...]
    out[...] = jnp.maximum(o, 0.0)


def _mlp(x, a0, a1, w1, b1, w2, b2):
    br = 1000
    full = pl.BlockSpec((D, D), lambda i: (0, 0))
    bias = pl.BlockSpec((1, D), lambda i: (0, 0))
    blk = pl.BlockSpec((br, D), lambda i: (i, 0))
    return pl.pallas_call(
        _mlp_body,
        grid=(N // br,),
        in_specs=[blk, blk, blk, full, bias, full, bias],
        out_specs=blk,
        out_shape=jax.ShapeDtypeStruct((N, D), jnp.float32),
    )(x, a0, a1, w1, b1.reshape(1, D), w2, b2.reshape(1, D))


_PBR = 1000


def _pool_body(h, batch3, wm1, bm1, wm2, bm2, out, acc_s, acc_c):
    i = pl.program_id(0)

    @pl.when(i == 0)
    def _init():
        acc_s[...] = jnp.zeros_like(acc_s)
        acc_c[...] = jnp.zeros_like(acc_c)

    b = batch3[0]
    oh = (lax.broadcasted_iota(jnp.int32, (G, _PBR), 0) == b
          ).astype(jnp.float32)
    acc_s[...] += jnp.dot(oh, h[...], preferred_element_type=jnp.float32)
    acc_c[...] += jnp.dot(oh, jnp.ones((_PBR, D), jnp.float32),
                          preferred_element_type=jnp.float32)

    @pl.when(i == pl.num_programs(0) - 1)
    def _fin():
        pooled = acc_s[...] / jnp.maximum(acc_c[...], 1.0)
        t = jnp.maximum(
            jnp.dot(pooled, wm1[...], preferred_element_type=jnp.float32)
            + bm1[...], 0.0)
        out[...] = jnp.dot(t, wm2[...],
                           preferred_element_type=jnp.float32) + bm2[...]


def _pool_classify(h, batch, wm1, bm1, wm2, bm2):
    nb = N // _PBR
    return pl.pallas_call(
        _pool_body,
        grid=(nb,),
        in_specs=[
            pl.BlockSpec((_PBR, D), lambda i: (i, 0)),
            pl.BlockSpec((1, 1, _PBR), lambda i: (i, 0, 0)),
            pl.BlockSpec((D, H2), lambda i: (0, 0)),
            pl.BlockSpec((1, H2), lambda i: (0, 0)),
            pl.BlockSpec((H2, NOUT), lambda i: (0, 0)),
            pl.BlockSpec((1, NOUT), lambda i: (0, 0)),
        ],
        out_specs=pl.BlockSpec((G, NOUT), lambda i: (0, 0)),
        out_shape=jax.ShapeDtypeStruct((G, NOUT), jnp.float32),
        scratch_shapes=[
            pltpu.VMEM((G, D), jnp.float32),
            pltpu.VMEM((G, D), jnp.float32),
        ],
    )(h, batch.reshape(nb, 1, _PBR), wm1, bm1.reshape(1, H2),
      wm2, bm2.reshape(1, NOUT))


def kernel(x, edge_index, edge_attr, batch, We1, be1, W11, b11, W12, b12,
           We2, be2, W21, b21, W22, b22, Wm1, bm1, Wm2, bm2):
    src = edge_index[0]
    dst = edge_index[1]
    e1, e2 = _edge_transform(edge_attr, We1, be1, We2, be2)
    p1 = _agg(x, e1, src, dst)
    h1 = _mlp(x, p1[:N], p1[N:], W11, b11, W12, b12)
    p2 = _agg(h1, e2, src, dst)
    h2 = _mlp(h1, p2[:N], p2[N:], W21, b21, W22, b22)
    return _pool_classify(h2, batch, Wm1, bm1, Wm2, bm2)
